# baseline (speedup 1.0000x reference)
import functools

import jax
import jax.numpy as jnp
from jax import lax
from jax.experimental import pallas as pl
from jax.experimental.pallas import tpu as pltpu
from jax.experimental.pallas import tpu_sc as plsc

N_VOTERS = 32
N_COLS = 1_000_000
HALF = N_VOTERS // 2
LANES = 16

NUM_CORES = 2
NUM_SUBCORES = 16
NW = NUM_CORES * NUM_SUBCORES

BLK = 1600
N_BLOCKS = N_COLS // BLK
MAX_K = -(-N_BLOCKS // NW)


def _vote_body(in_hbm, out_hbm, in_buf, out_buf):
    wid = lax.axis_index("s") * NUM_CORES + lax.axis_index("c")

    def block_step(k, _):
        blk = k * NW + wid

        @pl.when(blk < N_BLOCKS)
        def _():
            base = blk * BLK
            pltpu.sync_copy(in_hbm.at[:, pl.ds(base, BLK)], in_buf)

            def col_step(j, _):
                off = j * LANES
                acc = in_buf[0, pl.ds(off, LANES)]
                for i in range(1, N_VOTERS):
                    acc = acc + in_buf[i, pl.ds(off, LANES)]
                out_buf[pl.ds(off, LANES)] = jnp.where(acc > HALF, 1, 0)
                return 0

            lax.fori_loop(0, BLK // LANES, col_step, 0)
            pltpu.sync_copy(out_buf, out_hbm.at[pl.ds(base, BLK)])

        return 0

    lax.fori_loop(0, MAX_K, block_step, 0)


@jax.jit
def kernel(inputs):
    mesh = plsc.VectorSubcoreMesh(core_axis_name="c", subcore_axis_name="s")
    f = pl.kernel(
        _vote_body,
        out_type=jax.ShapeDtypeStruct((N_COLS,), jnp.int32),
        mesh=mesh,
        scratch_types=[
            pltpu.VMEM((N_VOTERS, BLK), jnp.int32),
            pltpu.VMEM((BLK,), jnp.int32),
        ],
        compiler_params=pltpu.CompilerParams(use_tc_tiling_on_sc=False),
    )
    return f(inputs)


# speedup vs baseline: 29.4750x; 29.4750x over previous
import functools

import jax
import jax.numpy as jnp
from jax import lax
from jax.experimental import pallas as pl
from jax.experimental.pallas import tpu as pltpu
from jax.experimental.pallas import tpu_sc as plsc

N_VOTERS = 32
N_COLS = 1_000_000
HALF = N_VOTERS // 2
LANES = 16

NUM_CORES = 2
NUM_SUBCORES = 16
NW = NUM_CORES * NUM_SUBCORES

BLK_SC = 1600
SC_BLOCKS = 0
SC_COLS = SC_BLOCKS * BLK_SC
MAX_K = -(-SC_BLOCKS // NW) if SC_BLOCKS else 0

BLK_TC = 8192


def _sc_body(in_hbm, out_hbm, in_buf, out_buf):
    wid = lax.axis_index("s") * NUM_CORES + lax.axis_index("c")

    def block_step(k, _):
        blk = k * NW + wid

        @pl.when(blk < SC_BLOCKS)
        def _():
            base = blk * BLK_SC
            pltpu.sync_copy(in_hbm.at[:, pl.ds(base, BLK_SC)], in_buf)

            def col_step(j, _):
                off = j * LANES
                acc = in_buf[0, pl.ds(off, LANES)]
                for i in range(1, N_VOTERS):
                    acc = acc + in_buf[i, pl.ds(off, LANES)]
                out_buf[pl.ds(off, LANES)] = jnp.where(acc > HALF, 1, 0)
                return 0

            lax.fori_loop(0, BLK_SC // LANES, col_step, 0)
            pltpu.sync_copy(out_buf, out_hbm.at[pl.ds(base, BLK_SC)])

        return 0

    lax.fori_loop(0, MAX_K, block_step, 0)


def _sc_vote(inputs_sc):
    mesh = plsc.VectorSubcoreMesh(core_axis_name="c", subcore_axis_name="s")
    f = pl.kernel(
        _sc_body,
        out_type=jax.ShapeDtypeStruct((SC_COLS,), jnp.int32),
        mesh=mesh,
        scratch_types=[
            pltpu.VMEM((N_VOTERS, BLK_SC), jnp.int32),
            pltpu.VMEM((BLK_SC,), jnp.int32),
        ],
        compiler_params=pltpu.CompilerParams(use_tc_tiling_on_sc=False),
    )
    return f(inputs_sc)


def _tc_body(in_ref, out_ref):
    s = jnp.sum(in_ref[...], axis=0)
    out_ref[...] = (s > HALF).astype(jnp.int32)


def _tc_vote(inputs_tc):
    n = inputs_tc.shape[1]
    grid = -(-n // BLK_TC)
    return pl.pallas_call(
        _tc_body,
        grid=(grid,),
        in_specs=[pl.BlockSpec((N_VOTERS, BLK_TC), lambda i: (0, i))],
        out_specs=pl.BlockSpec((BLK_TC,), lambda i: (i,)),
        out_shape=jax.ShapeDtypeStruct((n,), jnp.int32),
    )(inputs_tc)


@jax.jit
def kernel(inputs):
    if SC_COLS == 0:
        return _tc_vote(inputs)
    sc_out = _sc_vote(inputs[:, :SC_COLS])
    tc_out = _tc_vote(inputs[:, SC_COLS:])
    return jnp.concatenate([sc_out, tc_out])
